# baseline (speedup 1.0000x reference)
import functools

import jax
import jax.numpy as jnp
import numpy as np
from jax.experimental import pallas as pl

_N_MUT = 1024
_LB = 4096


@functools.lru_cache(maxsize=None)
def _tables(length: int):
    with jax.ensure_compile_time_eval():
        kp = jax.random.key(42)
        kpos, kperm = jax.random.split(kp)
        pos = np.asarray(jax.random.randint(kpos, (_N_MUT,), 0, length))
        perm_keys = jax.random.split(kperm, _N_MUT)
        perm = np.asarray(
            jax.vmap(lambda k: jax.random.permutation(k, 4))(perm_keys).T)
    g = np.tile(np.arange(4, dtype=np.int32)[:, None], (1, length))
    g[:, pos] = perm.astype(np.int32)
    return g


def _mutate_kernel(g_ref, h_ref, seq_ref, out_ref, rc_ref):
    s = seq_ref[...]
    gb = jnp.broadcast_to(g_ref[...], s.shape)
    out_ref[...] = jnp.take_along_axis(s, gb, axis=1)
    row = jax.lax.broadcasted_iota(jnp.int32, (128, 128), 0)
    col = jax.lax.broadcasted_iota(jnp.int32, (128, 128), 1)
    exch = (row + col == 127).astype(jnp.float32)
    nk = _LB // 128
    h = h_ref[...]
    for k in range(nk):
        t = nk - 1 - k
        sc = jax.lax.dot_general(
            s[:, :, k * 128:(k + 1) * 128], exch,
            (((2,), (0,)), ((), ())), preferred_element_type=jnp.float32,
            precision=jax.lax.Precision.HIGHEST)
        hb = jnp.broadcast_to(h[:, :, t * 128:(t + 1) * 128],
                              sc.shape)
        rc_ref[:, :, t * 128:(t + 1) * 128] = jnp.take_along_axis(
            sc, hb, axis=1)


def kernel(seq, rc, expression):
    del rc
    B, C, L = seq.shape
    assert C == 4 and L % _LB == 0
    g_np = _tables(L)
    h_np = g_np[::-1, ::-1].copy()
    g = jnp.asarray(g_np).reshape(1, C, L)
    h = jnp.asarray(h_np).reshape(1, C, L)
    nj = L // _LB
    out_seq, out_rc = pl.pallas_call(
        _mutate_kernel,
        grid=(nj,),
        in_specs=[
            pl.BlockSpec((1, C, _LB), lambda j: (0, 0, j)),
            pl.BlockSpec((1, C, _LB), lambda j: (0, 0, nj - 1 - j)),
            pl.BlockSpec((B, C, _LB), lambda j: (0, 0, j)),
        ],
        out_specs=[
            pl.BlockSpec((B, C, _LB), lambda j: (0, 0, j)),
            pl.BlockSpec((B, C, _LB), lambda j: (0, 0, nj - 1 - j)),
        ],
        out_shape=[
            jax.ShapeDtypeStruct(seq.shape, seq.dtype),
            jax.ShapeDtypeStruct(seq.shape, seq.dtype),
        ],
    )(g, h, seq)
    return (out_seq, out_rc, expression)


# speedup vs baseline: 1.2038x; 1.2038x over previous
import functools

import jax
import jax.numpy as jnp
import numpy as np
from jax.experimental import pallas as pl

_N_MUT = 1024
_LB = 4096


@functools.lru_cache(maxsize=None)
def _tables(length: int):
    with jax.ensure_compile_time_eval():
        kp = jax.random.key(42)
        kpos, kperm = jax.random.split(kp)
        pos = np.asarray(jax.random.randint(kpos, (_N_MUT,), 0, length))
        perm_keys = jax.random.split(kperm, _N_MUT)
        perm = np.asarray(
            jax.vmap(lambda k: jax.random.permutation(k, 4))(perm_keys).T)
    g = np.tile(np.arange(4, dtype=np.int32)[:, None], (1, length))
    g[:, pos] = perm.astype(np.int32)
    return g


def _mutate_kernel(g_ref, h_ref, seq_ref, out_ref, rc_ref):
    s = seq_ref[...]
    gb = jnp.broadcast_to(g_ref[...], s.shape)
    out_ref[...] = jnp.take_along_axis(s, gb, axis=1)
    row = jax.lax.broadcasted_iota(jnp.int32, (128, 128), 0)
    col = jax.lax.broadcasted_iota(jnp.int32, (128, 128), 1)
    exch = (row + col == 127).astype(jnp.float32)
    nk = _LB // 128
    h = h_ref[...]
    for k in range(nk):
        t = nk - 1 - k
        sc = jax.lax.dot_general(
            s[:, :, k * 128:(k + 1) * 128], exch,
            (((2,), (0,)), ((), ())), preferred_element_type=jnp.float32)
        hb = jnp.broadcast_to(h[:, :, t * 128:(t + 1) * 128],
                              sc.shape)
        rc_ref[:, :, t * 128:(t + 1) * 128] = jnp.take_along_axis(
            sc, hb, axis=1)


def kernel(seq, rc, expression):
    del rc
    B, C, L = seq.shape
    assert C == 4 and L % _LB == 0
    g_np = _tables(L)
    h_np = g_np[::-1, ::-1].copy()
    g = jnp.asarray(g_np).reshape(1, C, L)
    h = jnp.asarray(h_np).reshape(1, C, L)
    nj = L // _LB
    out_seq, out_rc = pl.pallas_call(
        _mutate_kernel,
        grid=(nj,),
        in_specs=[
            pl.BlockSpec((1, C, _LB), lambda j: (0, 0, j)),
            pl.BlockSpec((1, C, _LB), lambda j: (0, 0, nj - 1 - j)),
            pl.BlockSpec((B, C, _LB), lambda j: (0, 0, j)),
        ],
        out_specs=[
            pl.BlockSpec((B, C, _LB), lambda j: (0, 0, j)),
            pl.BlockSpec((B, C, _LB), lambda j: (0, 0, nj - 1 - j)),
        ],
        out_shape=[
            jax.ShapeDtypeStruct(seq.shape, seq.dtype),
            jax.ShapeDtypeStruct(seq.shape, seq.dtype),
        ],
    )(g, h, seq)
    return (out_seq, out_rc, expression)


# speedup vs baseline: 1.2052x; 1.0012x over previous
import functools

import jax
import jax.numpy as jnp
import numpy as np
from jax.experimental import pallas as pl

_N_MUT = 1024
_LB = 4096


@functools.lru_cache(maxsize=None)
def _tables(length: int):
    with jax.ensure_compile_time_eval():
        kp = jax.random.key(42)
        kpos, kperm = jax.random.split(kp)
        pos = np.asarray(jax.random.randint(kpos, (_N_MUT,), 0, length))
        perm_keys = jax.random.split(kperm, _N_MUT)
        perm = np.asarray(
            jax.vmap(lambda k: jax.random.permutation(k, 4))(perm_keys).T)
    g = np.tile(np.arange(4, dtype=np.int32)[:, None], (1, length))
    g[:, pos] = perm.astype(np.int32)
    return g


def _mutate_kernel(g_ref, h_ref, seq_ref, out_ref, rc_ref):
    s = seq_ref[...]
    g = g_ref[...]
    h = h_ref[...]
    row = jax.lax.broadcasted_iota(jnp.int32, (128, 128), 0)
    col = jax.lax.broadcasted_iota(jnp.int32, (128, 128), 1)
    exch = (row + col == 127).astype(jnp.float32)
    nk = _LB // 128
    for k in range(nk):
        t = nk - 1 - k
        sk = s[:, :, k * 128:(k + 1) * 128]
        gb = jnp.broadcast_to(g[:, :, k * 128:(k + 1) * 128], sk.shape)
        out_ref[:, :, k * 128:(k + 1) * 128] = jnp.take_along_axis(
            sk, gb, axis=1)
        sc = jax.lax.dot_general(
            sk, exch,
            (((2,), (0,)), ((), ())), preferred_element_type=jnp.float32)
        hb = jnp.broadcast_to(h[:, :, t * 128:(t + 1) * 128], sc.shape)
        rc_ref[:, :, t * 128:(t + 1) * 128] = jnp.take_along_axis(
            sc, hb, axis=1)


def kernel(seq, rc, expression):
    del rc
    B, C, L = seq.shape
    assert C == 4 and L % _LB == 0
    g_np = _tables(L)
    h_np = g_np[::-1, ::-1].copy()
    g = jnp.asarray(g_np).reshape(1, C, L)
    h = jnp.asarray(h_np).reshape(1, C, L)
    nj = L // _LB
    out_seq, out_rc = pl.pallas_call(
        _mutate_kernel,
        grid=(nj,),
        in_specs=[
            pl.BlockSpec((1, C, _LB), lambda j: (0, 0, j)),
            pl.BlockSpec((1, C, _LB), lambda j: (0, 0, nj - 1 - j)),
            pl.BlockSpec((B, C, _LB), lambda j: (0, 0, j)),
        ],
        out_specs=[
            pl.BlockSpec((B, C, _LB), lambda j: (0, 0, j)),
            pl.BlockSpec((B, C, _LB), lambda j: (0, 0, nj - 1 - j)),
        ],
        out_shape=[
            jax.ShapeDtypeStruct(seq.shape, seq.dtype),
            jax.ShapeDtypeStruct(seq.shape, seq.dtype),
        ],
    )(g, h, seq)
    return (out_seq, out_rc, expression)
